# baseline (speedup 1.0000x reference)
import functools

import jax
import jax.numpy as jnp
from jax import lax
from jax.experimental import pallas as pl
from jax.experimental.pallas import tpu as pltpu
from jax.experimental.pallas import tpu_sc as plsc

N_NODES = 10000
N_HEDGES = 10000
NNZ = 320000
D = 128
G = 8
CW = 8

NC, NS = 2, 16
NW = NC * NS
K = 80
NP = 10016
EPT = 10240
NNZ_P = NW * EPT
NBLK = EPT // K
RPT = NP // NS

RB = 400
NRB = N_NODES // RB


@functools.cache
def _make_sc_pass(with_counts):
    mesh = plsc.VectorSubcoreMesh(
        core_axis_name="c", subcore_axis_name="s", num_cores=NC, num_subcores=NS)

    if with_counts:
        out_type = (
            jax.ShapeDtypeStruct((NC, NP, D), jnp.float32),
            jax.ShapeDtypeStruct((NC, NP, CW), jnp.float32),
        )
        extra = [
            pltpu.VMEM((K, CW), jnp.float32),
            pltpu.VMEM_SHARED((NP, CW), jnp.float32),
            pltpu.SemaphoreType.DMA,
            pltpu.SemaphoreType.DMA,
        ]
    else:
        out_type = jax.ShapeDtypeStruct((NC, NP, D), jnp.float32)
        extra = []

    @functools.partial(
        pl.kernel,
        out_type=out_type,
        mesh=mesh,
        scratch_types=[
            pltpu.VMEM((8, 2, K), jnp.int32),
            pltpu.VMEM((K, D), jnp.float32),
            pltpu.VMEM((K, D), jnp.float32),
            pltpu.VMEM((K, D), jnp.float32),
            pltpu.VMEM((K, D), jnp.float32),
        ] + extra + [
            pltpu.VMEM_SHARED((NP, D), jnp.float32),
        ] + [pltpu.SemaphoreType.DMA] * 16,
        compiler_params=pltpu.CompilerParams(use_tc_tiling_on_sc=False),
    )
    def _sc_pass(table, idx, zeros, zeros_c, ones, *rest):
        if with_counts:
            (out, out_c, ring, b0, b1, b2, b3, ones_v, acc_c, os0, os1,
             acc, *sems) = rest
            osem = (os0, os1)
        else:
            (out, ring, b0, b1, b2, b3, acc, *sems) = rest
        gsem = tuple(sems[0:4])
        ssem = tuple(sems[4:8])
        isem = tuple(sems[8:16])
        bufs = (b0, b1, b2, b3)
        c = lax.axis_index("c")
        s = lax.axis_index("s")
        w = c * NS + s

        pltpu.sync_copy(idx.at[w, 0], ring.at[0])
        pltpu.sync_copy(idx.at[w, 1], ring.at[1])
        for t in (2, 3, 4, 5):
            pltpu.async_copy(idx.at[w, t], ring.at[t], isem[t])
        pltpu.async_copy(table.at[ring.at[0, 0]], bufs[0], gsem[0])
        pltpu.async_copy(table.at[ring.at[1, 0]], bufs[1], gsem[1])
        pltpu.sync_copy(zeros, acc.at[pl.ds(s * RPT, RPT)])
        if with_counts:
            pltpu.sync_copy(ones, ones_v)
            pltpu.sync_copy(zeros_c, acc_c.at[pl.ds(s * RPT, RPT)])
        plsc.subcore_barrier()

        def body(i, carry):
            j = i * 8
            for p in range(8):
                b = j + p
                X = p % 4
                R = p
                rbuf = bufs[X]
                pltpu.make_async_copy(
                    table.at[ring.at[R, 0]], rbuf, gsem[X]).wait()
                if with_counts:
                    @pl.when(b >= 2)
                    def _():
                        pltpu.make_async_copy(
                            ones_v, acc_c.at[ring.at[R, 1]],
                            osem[p % 2]).wait()
                pltpu.async_copy(rbuf, acc.at[ring.at[R, 1]], ssem[X],
                                 add=True)
                if with_counts:
                    pltpu.async_copy(ones_v, acc_c.at[ring.at[R, 1]],
                                     osem[p % 2], add=True)

                @pl.when(b >= 2)
                def _():
                    pltpu.make_async_copy(
                        bufs[(X + 2) % 4], acc.at[ring.at[R, 1]],
                        ssem[(X + 2) % 4]).wait()

                @pl.when(b + 2 < NBLK)
                def _():
                    pltpu.make_async_copy(
                        idx.at[w, 0], ring.at[(R + 2) % 8],
                        isem[(R + 2) % 8]).wait()
                    pltpu.async_copy(
                        table.at[ring.at[(R + 2) % 8, 0]],
                        bufs[(X + 2) % 4], gsem[(X + 2) % 4])

                @pl.when(b + 6 < NBLK)
                def _():
                    pltpu.async_copy(
                        idx.at[w, b + 6], ring.at[(R + 6) % 8],
                        isem[(R + 6) % 8])

            return carry

        lax.fori_loop(0, NBLK // 8, body, 0)
        pltpu.make_async_copy(bufs[2], acc.at[ring.at[6, 1]], ssem[2]).wait()
        pltpu.make_async_copy(bufs[3], acc.at[ring.at[7, 1]], ssem[3]).wait()
        if with_counts:
            pltpu.make_async_copy(
                ones_v, acc_c.at[ring.at[6, 1]], osem[0]).wait()
            pltpu.make_async_copy(
                ones_v, acc_c.at[ring.at[7, 1]], osem[1]).wait()
        plsc.subcore_barrier()
        pltpu.sync_copy(acc.at[pl.ds(s * RPT, RPT)],
                        out.at[c, pl.ds(s * RPT, RPT)])
        if with_counts:
            pltpu.sync_copy(acc_c.at[pl.ds(s * RPT, RPT)],
                            out_c.at[c, pl.ds(s * RPT, RPT)])

    return _sc_pass


def _prep_body(x_ref, w0_ref, b0_ref, w1_ref, out_ref):
    h = jnp.maximum(
        jnp.dot(x_ref[...], w0_ref[...], preferred_element_type=jnp.float32)
        + b0_ref[...], 0.0)
    out_ref[...] = jnp.dot(h, w1_ref[...], preferred_element_type=jnp.float32)


def _safe_inv(v):
    return jnp.where(v > 0, 1.0 / jnp.where(v > 0, v, 1.0), 0.0)


def _combine_body(p_ref, pc_ref, out_ref):
    p = p_ref[0] + p_ref[1]
    cnt = (pc_ref[0] + pc_ref[1])[:, 0:1]
    out_ref[...] = p * _safe_inv(cnt)


def _post1_body(p_ref, pc_ref, bc_ref, w2_ref, batch_ref,
                y2_ref, pool_ref, cnt_ref):
    i = pl.program_id(0)
    p = p_ref[0] + p_ref[1]
    d = (pc_ref[0] + pc_ref[1])[:, 0:1]
    h = jnp.maximum(p * _safe_inv(d) + bc_ref[...], 0.0)
    y2_ref[...] = jnp.dot(h, w2_ref[...], preferred_element_type=jnp.float32)
    b = batch_ref[0]
    gi = lax.broadcasted_iota(jnp.int32, (G, RB), 0)
    oh = (gi == b).astype(jnp.float32)

    @pl.when(i == 0)
    def _():
        pool_ref[...] = jnp.zeros_like(pool_ref)
        cnt_ref[...] = jnp.zeros_like(cnt_ref)

    pool_ref[...] += jnp.dot(oh, h, preferred_element_type=jnp.float32)
    cnt_ref[...] += jnp.sum(oh, axis=1, keepdims=True)


def _post2_body(p_ref, pc_ref, bc_ref, batch_ref, p1_ref, c_ref, wl_ref,
                bl_ref, w1_ref, b1_ref, g1_ref, be1_ref, w2_ref, b2_ref,
                g2_ref, be2_ref, wo_ref, pool_ref, out_ref):
    i = pl.program_id(0)
    p = p_ref[0] + p_ref[1]
    d = (pc_ref[0] + pc_ref[1])[:, 0:1]
    h = jnp.maximum(p * _safe_inv(d) + bc_ref[...], 0.0)
    b = batch_ref[0]
    gi = lax.broadcasted_iota(jnp.int32, (G, RB), 0)
    oh = (gi == b).astype(jnp.float32)

    @pl.when(i == 0)
    def _():
        pool_ref[...] = jnp.zeros_like(pool_ref)

    pool_ref[...] += jnp.dot(oh, h, preferred_element_type=jnp.float32)

    @pl.when(i == NRB - 1)
    def _():
        cnt = jnp.maximum(c_ref[...], 1.0)
        p1 = p1_ref[...] / cnt
        p2 = pool_ref[...] / cnt
        glob = jnp.concatenate([p1, p2], axis=1)
        he = (jnp.dot(glob, wl_ref[...], preferred_element_type=jnp.float32)
              + bl_ref[...])
        s = 0.9999950000374997
        m = jnp.maximum(
            (jnp.dot(he, w1_ref[...], preferred_element_type=jnp.float32)
             + b1_ref[...]) * s * g1_ref[...] + be1_ref[...], 0.0)
        m = jnp.maximum(
            (jnp.dot(m, w2_ref[...], preferred_element_type=jnp.float32)
             + b2_ref[...]) * s * g2_ref[...] + be2_ref[...], 0.0)
        out_ref[...] = jnp.dot(m, wo_ref[...], preferred_element_type=jnp.float32)


def _full(shape):
    return pl.BlockSpec(shape, lambda i: tuple(0 for _ in shape))


_prep = pl.pallas_call(
    _prep_body,
    grid=(NRB,),
    in_specs=[
        pl.BlockSpec((RB, D), lambda i: (i, 0)),
        _full((D, D)),
        _full((1, D)),
        _full((D, D)),
    ],
    out_specs=pl.BlockSpec((RB, D), lambda i: (i, 0)),
    out_shape=jax.ShapeDtypeStruct((N_NODES, D), jnp.float32),
)

_combine = pl.pallas_call(
    _combine_body,
    grid=(NRB,),
    in_specs=[
        pl.BlockSpec((NC, RB, D), lambda i: (0, i, 0)),
        pl.BlockSpec((NC, RB, CW), lambda i: (0, i, 0)),
    ],
    out_specs=pl.BlockSpec((RB, D), lambda i: (i, 0)),
    out_shape=jax.ShapeDtypeStruct((N_NODES, D), jnp.float32),
)

_post1 = pl.pallas_call(
    _post1_body,
    grid=(NRB,),
    in_specs=[
        pl.BlockSpec((NC, RB, D), lambda i: (0, i, 0)),
        pl.BlockSpec((NC, RB, CW), lambda i: (0, i, 0)),
        _full((1, D)),
        _full((D, D)),
        pl.BlockSpec((1, 1, RB), lambda i: (i, 0, 0)),
    ],
    out_specs=(
        pl.BlockSpec((RB, D), lambda i: (i, 0)),
        pl.BlockSpec((G, D), lambda i: (0, 0)),
        pl.BlockSpec((G, 1), lambda i: (0, 0)),
    ),
    out_shape=(
        jax.ShapeDtypeStruct((N_NODES, D), jnp.float32),
        jax.ShapeDtypeStruct((G, D), jnp.float32),
        jax.ShapeDtypeStruct((G, 1), jnp.float32),
    ),
)

_post2 = pl.pallas_call(
    _post2_body,
    grid=(NRB,),
    in_specs=[
        pl.BlockSpec((NC, RB, D), lambda i: (0, i, 0)),
        pl.BlockSpec((NC, RB, CW), lambda i: (0, i, 0)),
        _full((1, D)),
        pl.BlockSpec((1, 1, RB), lambda i: (i, 0, 0)),
        _full((G, D)),
        _full((G, 1)),
        _full((256, D)),
        _full((1, D)),
        _full((D, 64)),
        _full((1, 64)),
        _full((1, 64)),
        _full((1, 64)),
        _full((64, 32)),
        _full((1, 32)),
        _full((1, 32)),
        _full((1, 32)),
        _full((32, 4)),
    ],
    out_specs=(
        pl.BlockSpec((G, D), lambda i: (0, 0)),
        pl.BlockSpec((G, 4), lambda i: (0, 0)),
    ),
    out_shape=(
        jax.ShapeDtypeStruct((G, D), jnp.float32),
        jax.ShapeDtypeStruct((G, 4), jnp.float32),
    ),
)


def kernel(x, edge_index, batch, W_first, b_first, Wc1, bc1, Wc2, bc2, W_lin,
           b_lin, Wm1, bm1, g1, be1, Wm2, bm2, g2, be2, Wout):
    f32 = jnp.float32
    node_idx = edge_index[0]
    hedge_idx = edge_index[1]

    npad = NNZ_P - NNZ
    pad_g = (jnp.arange(npad, dtype=jnp.int32) * 41) % N_NODES
    pad_s = N_HEDGES + jnp.arange(npad, dtype=jnp.int32) % (NP - N_HEDGES)
    gidx1 = jnp.concatenate([node_idx, pad_g]).reshape(NW, NBLK, K)
    sidx1 = jnp.concatenate([hedge_idx, pad_s]).reshape(NW, NBLK, K)
    gidx2 = jnp.concatenate([hedge_idx, pad_g]).reshape(NW, NBLK, K)
    sidx2 = jnp.concatenate([node_idx, pad_s]).reshape(NW, NBLK, K)
    idx1 = jnp.stack([gidx1, sidx1], axis=2)
    idx2 = jnp.stack([gidx2, sidx2], axis=2)

    zeros = jnp.zeros((RPT, D), f32)
    zeros_c = jnp.zeros((RPT, CW), f32)
    ones = jnp.ones((K, CW), f32)
    batch_r = batch.reshape(NRB, 1, RB)

    sc_c = _make_sc_pass(True)
    sc_n = _make_sc_pass(False)
    y1 = _prep(x, W_first, b_first.reshape(1, D), Wc1)
    pm1, cb = sc_c(y1, idx1, zeros, zeros_c, ones)
    m1 = _combine(pm1, cb)
    po1, cd = sc_c(m1, idx2, zeros, zeros_c, ones)
    y2, pool1, cntg = _post1(po1, cd, bc1.reshape(1, D), Wc2, batch_r)
    pm2 = sc_n(y2, idx1, zeros, zeros_c, ones)
    m2 = _combine(pm2, cb)
    po2 = sc_n(m2, idx2, zeros, zeros_c, ones)
    _pool2, out = _post2(po2, cd, bc2.reshape(1, D), batch_r, pool1, cntg,
                         W_lin, b_lin.reshape(1, D), Wm1, bm1.reshape(1, 64),
                         g1.reshape(1, 64), be1.reshape(1, 64), Wm2,
                         bm2.reshape(1, 32), g2.reshape(1, 32),
                         be2.reshape(1, 32), Wout)
    return out


# speedup vs baseline: 1.0808x; 1.0808x over previous
import functools

import jax
import jax.numpy as jnp
from jax import lax
from jax.experimental import pallas as pl
from jax.experimental.pallas import tpu as pltpu
from jax.experimental.pallas import tpu_sc as plsc

N_NODES = 10000
N_HEDGES = 10000
NNZ = 320000
D = 128
G = 8
CW = 16

NC, NS = 2, 16
NW = NC * NS
K = 128
NP = 10240
EPT = NP
NNZ_P = NW * EPT
NBLK = EPT // K
RPT = NP // NS

RB = 400
NRB = N_NODES // RB

NP2 = 10016
NBLK2 = 84
EPT2 = NBLK2 * K
NNZ_P2 = NW * EPT2
RPT2 = NP2 // NS


def _make_sc_plain(mesh):
    @functools.partial(
        pl.kernel,
        out_type=jax.ShapeDtypeStruct((NC, NP2, D), jnp.float32),
        mesh=mesh,
        scratch_types=[
            pltpu.VMEM((6, 2, K), jnp.int32),
            pltpu.VMEM((K, D), jnp.float32),
            pltpu.VMEM((K, D), jnp.float32),
            pltpu.VMEM((K, D), jnp.float32),
            pltpu.VMEM_SHARED((NP2, D), jnp.float32),
        ] + [pltpu.SemaphoreType.DMA] * 12,
        compiler_params=pltpu.CompilerParams(use_tc_tiling_on_sc=False),
    )
    def _sc_plain(table, idx, zeros, out, ring, b0, b1, b2, acc, *sems):
        gsem = tuple(sems[0:3])
        ssem = tuple(sems[3:6])
        isem = tuple(sems[6:12])
        bufs = (b0, b1, b2)
        c = lax.axis_index("c")
        s = lax.axis_index("s")
        w = c * NS + s

        pltpu.sync_copy(idx.at[w, 0], ring.at[0])
        pltpu.sync_copy(idx.at[w, 1], ring.at[1])
        pltpu.async_copy(idx.at[w, 2], ring.at[2], isem[2])
        pltpu.async_copy(idx.at[w, 3], ring.at[3], isem[3])
        pltpu.async_copy(table.at[ring.at[0, 0]], bufs[0], gsem[0])
        pltpu.async_copy(table.at[ring.at[1, 0]], bufs[1], gsem[1])
        pltpu.sync_copy(zeros, acc.at[pl.ds(s * RPT2, RPT2)])
        plsc.subcore_barrier()

        def body(i, carry):
            j = i * 6
            for p in range(6):
                b = j + p
                X = p % 3
                R = p
                pltpu.make_async_copy(
                    table.at[ring.at[R, 0]], bufs[X], gsem[X]).wait()
                pltpu.async_copy(bufs[X], acc.at[ring.at[R, 1]], ssem[X],
                                 add=True)

                @pl.when(b >= 1)
                def _():
                    pltpu.make_async_copy(
                        bufs[(X + 2) % 3], acc.at[ring.at[R, 1]],
                        ssem[(X + 2) % 3]).wait()

                @pl.when(b + 2 < NBLK2)
                def _():
                    pltpu.make_async_copy(
                        idx.at[w, 0], ring.at[(R + 2) % 6],
                        isem[(R + 2) % 6]).wait()
                    pltpu.async_copy(
                        table.at[ring.at[(R + 2) % 6, 0]],
                        bufs[(X + 2) % 3], gsem[(X + 2) % 3])

                @pl.when(b + 4 < NBLK2)
                def _():
                    pltpu.async_copy(
                        idx.at[w, b + 4], ring.at[(R + 4) % 6],
                        isem[(R + 4) % 6])

            return carry

        lax.fori_loop(0, NBLK2 // 6, body, 0)
        pltpu.make_async_copy(
            bufs[(NBLK2 - 1) % 3], acc.at[ring.at[5, 1]],
            ssem[(NBLK2 - 1) % 3]).wait()
        plsc.subcore_barrier()
        pltpu.sync_copy(acc.at[pl.ds(s * RPT2, RPT2)],
                        out.at[c, pl.ds(s * RPT2, RPT2)])

    return _sc_plain


@functools.cache
def _make_sc_pass(with_counts):
    mesh = plsc.VectorSubcoreMesh(
        core_axis_name="c", subcore_axis_name="s", num_cores=NC, num_subcores=NS)

    if not with_counts:
        return _make_sc_plain(mesh)
    out_type = (
        jax.ShapeDtypeStruct((NC, NP, D), jnp.float32),
        jax.ShapeDtypeStruct((NC, NP, CW), jnp.float32),
    )
    extra = [
        pltpu.VMEM((K, CW), jnp.float32),
        pltpu.VMEM_SHARED((NP, CW), jnp.float32),
        pltpu.SemaphoreType.DMA,
    ]
    with_counts = True

    @functools.partial(
        pl.kernel,
        out_type=out_type,
        mesh=mesh,
        scratch_types=[
            pltpu.VMEM((4, 2, K), jnp.int32),
            pltpu.VMEM((K, D), jnp.float32),
            pltpu.VMEM((K, D), jnp.float32),
        ] + extra + [
            pltpu.VMEM_SHARED((NP, D), jnp.float32),
            pltpu.SemaphoreType.DMA,
            pltpu.SemaphoreType.DMA,
            pltpu.SemaphoreType.DMA,
            pltpu.SemaphoreType.DMA,
            pltpu.SemaphoreType.DMA,
            pltpu.SemaphoreType.DMA,
        ],
        compiler_params=pltpu.CompilerParams(use_tc_tiling_on_sc=False),
    )
    def _sc_pass(table, idx, zeros, zeros_c, ones, *rest):
        if with_counts:
            (out, out_c, ring, rows0, rows1, ones_v, acc_c, sem_o, acc,
             sem0, sem1, si0, si1, si2, si3) = rest
        else:
            (out, ring, rows0, rows1, acc,
             sem0, sem1, si0, si1, si2, si3) = rest
        c = lax.axis_index("c")
        s = lax.axis_index("s")
        w = c * NS + s
        sis = (si0, si1, si2, si3)

        pltpu.sync_copy(idx.at[w, 0], ring.at[0])
        pltpu.sync_copy(idx.at[w, 1], ring.at[1])
        pltpu.async_copy(idx.at[w, 2], ring.at[2], si2)
        pltpu.async_copy(idx.at[w, 3], ring.at[3], si3)
        pltpu.async_copy(table.at[ring.at[0, 0]], rows0, sem0)
        pltpu.async_copy(table.at[ring.at[1, 0]], rows1, sem1)
        pltpu.sync_copy(zeros, acc.at[pl.ds(s * RPT, RPT)])
        if with_counts:
            pltpu.sync_copy(ones, ones_v)
            pltpu.sync_copy(zeros_c, acc_c.at[pl.ds(s * RPT, RPT)])
        plsc.subcore_barrier()

        def body(i, carry):
            j = i * 4
            for p in range(4):
                b = j + p
                rbuf = rows0 if p % 2 == 0 else rows1
                rsem = sem0 if p % 2 == 0 else sem1
                gslot = (p + 2) % 4
                pltpu.make_async_copy(
                    table.at[ring.at[p, 0]], rbuf, rsem).wait()
                if with_counts:
                    pltpu.async_copy(
                        ones_v, acc_c.at[ring.at[p, 1]], sem_o, add=True)
                pltpu.sync_copy(rbuf, acc.at[ring.at[p, 1]], add=True)
                if with_counts:
                    pltpu.make_async_copy(
                        ones_v, acc_c.at[ring.at[p, 1]], sem_o).wait()

                @pl.when(b + 4 < NBLK)
                def _():
                    pltpu.async_copy(idx.at[w, b + 4], ring.at[p], sis[p])

                @pl.when(b + 2 < NBLK)
                def _():
                    pltpu.make_async_copy(
                        idx.at[w, b + 2], ring.at[gslot], sis[gslot]).wait()
                    pltpu.async_copy(
                        table.at[ring.at[gslot, 0]], rbuf, rsem)

            return carry

        lax.fori_loop(0, NBLK // 4, body, 0)
        plsc.subcore_barrier()
        pltpu.sync_copy(acc.at[pl.ds(s * RPT, RPT)],
                        out.at[c, pl.ds(s * RPT, RPT)])
        if with_counts:
            pltpu.sync_copy(acc_c.at[pl.ds(s * RPT, RPT)],
                            out_c.at[c, pl.ds(s * RPT, RPT)])

    return _sc_pass


def _prep_body(x_ref, w0_ref, b0_ref, w1_ref, out_ref):
    h = jnp.maximum(
        jnp.dot(x_ref[...], w0_ref[...], preferred_element_type=jnp.float32)
        + b0_ref[...], 0.0)
    out_ref[...] = jnp.dot(h, w1_ref[...], preferred_element_type=jnp.float32)


def _safe_inv(v):
    return jnp.where(v > 0, 1.0 / jnp.where(v > 0, v, 1.0), 0.0)


def _combine_body(p_ref, pc_ref, out_ref):
    p = p_ref[0] + p_ref[1]
    cnt = (pc_ref[0] + pc_ref[1])[:, 0:1]
    out_ref[...] = p * _safe_inv(cnt)


def _post1_body(p_ref, pc_ref, bc_ref, w2_ref, batch_ref,
                y2_ref, pool_ref, cnt_ref):
    i = pl.program_id(0)
    p = p_ref[0] + p_ref[1]
    d = (pc_ref[0] + pc_ref[1])[:, 0:1]
    h = jnp.maximum(p * _safe_inv(d) + bc_ref[...], 0.0)
    y2_ref[...] = jnp.dot(h, w2_ref[...], preferred_element_type=jnp.float32)
    b = batch_ref[0]
    gi = lax.broadcasted_iota(jnp.int32, (G, RB), 0)
    oh = (gi == b).astype(jnp.float32)

    @pl.when(i == 0)
    def _():
        pool_ref[...] = jnp.zeros_like(pool_ref)
        cnt_ref[...] = jnp.zeros_like(cnt_ref)

    pool_ref[...] += jnp.dot(oh, h, preferred_element_type=jnp.float32)
    cnt_ref[...] += jnp.sum(oh, axis=1, keepdims=True)


def _post2_body(p_ref, pc_ref, bc_ref, batch_ref, p1_ref, c_ref, wl_ref,
                bl_ref, w1_ref, b1_ref, g1_ref, be1_ref, w2_ref, b2_ref,
                g2_ref, be2_ref, wo_ref, pool_ref, out_ref):
    i = pl.program_id(0)
    p = p_ref[0] + p_ref[1]
    d = (pc_ref[0] + pc_ref[1])[:, 0:1]
    h = jnp.maximum(p * _safe_inv(d) + bc_ref[...], 0.0)
    b = batch_ref[0]
    gi = lax.broadcasted_iota(jnp.int32, (G, RB), 0)
    oh = (gi == b).astype(jnp.float32)

    @pl.when(i == 0)
    def _():
        pool_ref[...] = jnp.zeros_like(pool_ref)

    pool_ref[...] += jnp.dot(oh, h, preferred_element_type=jnp.float32)

    @pl.when(i == NRB - 1)
    def _():
        cnt = jnp.maximum(c_ref[...], 1.0)
        p1 = p1_ref[...] / cnt
        p2 = pool_ref[...] / cnt
        glob = jnp.concatenate([p1, p2], axis=1)
        he = (jnp.dot(glob, wl_ref[...], preferred_element_type=jnp.float32)
              + bl_ref[...])
        s = 0.9999950000374997
        m = jnp.maximum(
            (jnp.dot(he, w1_ref[...], preferred_element_type=jnp.float32)
             + b1_ref[...]) * s * g1_ref[...] + be1_ref[...], 0.0)
        m = jnp.maximum(
            (jnp.dot(m, w2_ref[...], preferred_element_type=jnp.float32)
             + b2_ref[...]) * s * g2_ref[...] + be2_ref[...], 0.0)
        out_ref[...] = jnp.dot(m, wo_ref[...], preferred_element_type=jnp.float32)


def _full(shape):
    return pl.BlockSpec(shape, lambda i: tuple(0 for _ in shape))


_prep = pl.pallas_call(
    _prep_body,
    grid=(NRB,),
    in_specs=[
        pl.BlockSpec((RB, D), lambda i: (i, 0)),
        _full((D, D)),
        _full((1, D)),
        _full((D, D)),
    ],
    out_specs=pl.BlockSpec((RB, D), lambda i: (i, 0)),
    out_shape=jax.ShapeDtypeStruct((N_NODES, D), jnp.float32),
)

_combine = pl.pallas_call(
    _combine_body,
    grid=(NRB,),
    in_specs=[
        pl.BlockSpec((NC, RB, D), lambda i: (0, i, 0)),
        pl.BlockSpec((NC, RB, CW), lambda i: (0, i, 0)),
    ],
    out_specs=pl.BlockSpec((RB, D), lambda i: (i, 0)),
    out_shape=jax.ShapeDtypeStruct((N_NODES, D), jnp.float32),
)

_post1 = pl.pallas_call(
    _post1_body,
    grid=(NRB,),
    in_specs=[
        pl.BlockSpec((NC, RB, D), lambda i: (0, i, 0)),
        pl.BlockSpec((NC, RB, CW), lambda i: (0, i, 0)),
        _full((1, D)),
        _full((D, D)),
        pl.BlockSpec((1, 1, RB), lambda i: (i, 0, 0)),
    ],
    out_specs=(
        pl.BlockSpec((RB, D), lambda i: (i, 0)),
        pl.BlockSpec((G, D), lambda i: (0, 0)),
        pl.BlockSpec((G, 1), lambda i: (0, 0)),
    ),
    out_shape=(
        jax.ShapeDtypeStruct((N_NODES, D), jnp.float32),
        jax.ShapeDtypeStruct((G, D), jnp.float32),
        jax.ShapeDtypeStruct((G, 1), jnp.float32),
    ),
)

_post2 = pl.pallas_call(
    _post2_body,
    grid=(NRB,),
    in_specs=[
        pl.BlockSpec((NC, RB, D), lambda i: (0, i, 0)),
        pl.BlockSpec((NC, RB, CW), lambda i: (0, i, 0)),
        _full((1, D)),
        pl.BlockSpec((1, 1, RB), lambda i: (i, 0, 0)),
        _full((G, D)),
        _full((G, 1)),
        _full((256, D)),
        _full((1, D)),
        _full((D, 64)),
        _full((1, 64)),
        _full((1, 64)),
        _full((1, 64)),
        _full((64, 32)),
        _full((1, 32)),
        _full((1, 32)),
        _full((1, 32)),
        _full((32, 4)),
    ],
    out_specs=(
        pl.BlockSpec((G, D), lambda i: (0, 0)),
        pl.BlockSpec((G, 4), lambda i: (0, 0)),
    ),
    out_shape=(
        jax.ShapeDtypeStruct((G, D), jnp.float32),
        jax.ShapeDtypeStruct((G, 4), jnp.float32),
    ),
)


def kernel(x, edge_index, batch, W_first, b_first, Wc1, bc1, Wc2, bc2, W_lin,
           b_lin, Wm1, bm1, g1, be1, Wm2, bm2, g2, be2, Wout):
    f32 = jnp.float32
    node_idx = edge_index[0]
    hedge_idx = edge_index[1]

    npad = NNZ_P - NNZ
    pad_g = (jnp.arange(npad, dtype=jnp.int32) * 41) % N_NODES
    pad_s = N_HEDGES + jnp.arange(npad, dtype=jnp.int32) % (NP - N_HEDGES)
    gidx1 = jnp.concatenate([node_idx, pad_g]).reshape(NW, NBLK, K)
    sidx1 = jnp.concatenate([hedge_idx, pad_s]).reshape(NW, NBLK, K)
    gidx2 = jnp.concatenate([hedge_idx, pad_g]).reshape(NW, NBLK, K)
    sidx2 = jnp.concatenate([node_idx, pad_s]).reshape(NW, NBLK, K)
    idx1 = jnp.stack([gidx1, sidx1], axis=2)
    idx2 = jnp.stack([gidx2, sidx2], axis=2)

    npad2 = NNZ_P2 - NNZ
    pad_g2 = (jnp.arange(npad2, dtype=jnp.int32) * 41) % N_NODES
    pad_s2 = N_HEDGES + jnp.arange(npad2, dtype=jnp.int32) % (NP2 - N_HEDGES)
    idx1b = jnp.stack([
        jnp.concatenate([node_idx, pad_g2]).reshape(NW, NBLK2, K),
        jnp.concatenate([hedge_idx, pad_s2]).reshape(NW, NBLK2, K)], axis=2)
    idx2b = jnp.stack([
        jnp.concatenate([hedge_idx, pad_g2]).reshape(NW, NBLK2, K),
        jnp.concatenate([node_idx, pad_s2]).reshape(NW, NBLK2, K)], axis=2)

    zeros = jnp.zeros((RPT, D), f32)
    zeros2 = jnp.zeros((RPT2, D), f32)
    zeros_c = jnp.zeros((RPT, CW), f32)
    ones = jnp.ones((K, CW), f32)
    batch_r = batch.reshape(NRB, 1, RB)

    sc_c = _make_sc_pass(True)
    sc_n = _make_sc_pass(False)
    y1 = _prep(x, W_first, b_first.reshape(1, D), Wc1)
    pm1, cb = sc_c(y1, idx1, zeros, zeros_c, ones)
    m1 = _combine(pm1, cb)
    po1, cd = sc_c(m1, idx2, zeros, zeros_c, ones)
    y2, pool1, cntg = _post1(po1, cd, bc1.reshape(1, D), Wc2, batch_r)
    pm2 = sc_n(y2, idx1b, zeros2)
    m2 = _combine(pm2, cb)
    po2 = sc_n(m2, idx2b, zeros2)
    _pool2, out = _post2(po2, cd, bc2.reshape(1, D), batch_r, pool1, cntg,
                         W_lin, b_lin.reshape(1, D), Wm1, bm1.reshape(1, 64),
                         g1.reshape(1, 64), be1.reshape(1, 64), Wm2,
                         bm2.reshape(1, 32), g2.reshape(1, 32),
                         be2.reshape(1, 32), Wout)
    return out


# speedup vs baseline: 1.0862x; 1.0050x over previous
import functools

import jax
import jax.numpy as jnp
from jax import lax
from jax.experimental import pallas as pl
from jax.experimental.pallas import tpu as pltpu
from jax.experimental.pallas import tpu_sc as plsc

N_NODES = 10000
N_HEDGES = 10000
NNZ = 320000
D = 128
G = 8
CW = 16

NC, NS = 2, 16
NW = NC * NS
K = 128
NP = 10240
EPT = NP
NNZ_P = NW * EPT
NBLK = EPT // K
RPT = NP // NS

RB = 400
NRB = N_NODES // RB


@functools.cache
def _make_sc_pass(with_counts):
    mesh = plsc.VectorSubcoreMesh(
        core_axis_name="c", subcore_axis_name="s", num_cores=NC, num_subcores=NS)

    if with_counts:
        out_type = (
            jax.ShapeDtypeStruct((NC, NP, D), jnp.float32),
            jax.ShapeDtypeStruct((NC, NP, CW), jnp.float32),
        )
        extra = [
            pltpu.VMEM((K, CW), jnp.float32),
            pltpu.VMEM_SHARED((NP, CW), jnp.float32),
            pltpu.SemaphoreType.DMA,
        ]
    else:
        out_type = jax.ShapeDtypeStruct((NC, NP, D), jnp.float32)
        extra = []

    @functools.partial(
        pl.kernel,
        out_type=out_type,
        mesh=mesh,
        scratch_types=[
            pltpu.VMEM((4, 2, K), jnp.int32),
            pltpu.VMEM((K, D), jnp.float32),
            pltpu.VMEM((K, D), jnp.float32),
        ] + extra + [
            pltpu.VMEM_SHARED((NP, D), jnp.float32),
            pltpu.SemaphoreType.DMA,
            pltpu.SemaphoreType.DMA,
            pltpu.SemaphoreType.DMA,
            pltpu.SemaphoreType.DMA,
            pltpu.SemaphoreType.DMA,
            pltpu.SemaphoreType.DMA,
        ],
        compiler_params=pltpu.CompilerParams(use_tc_tiling_on_sc=False),
    )
    def _sc_pass(table, idx, zeros, zeros_c, ones, *rest):
        if with_counts:
            (out, out_c, ring, rows0, rows1, ones_v, acc_c, sem_o, acc,
             sem0, sem1, si0, si1, si2, si3) = rest
        else:
            (out, ring, rows0, rows1, acc,
             sem0, sem1, si0, si1, si2, si3) = rest
        c = lax.axis_index("c")
        s = lax.axis_index("s")
        w = c * NS + s
        sis = (si0, si1, si2, si3)

        pltpu.sync_copy(idx.at[w, 0], ring.at[0])
        pltpu.sync_copy(idx.at[w, 1], ring.at[1])
        pltpu.async_copy(idx.at[w, 2], ring.at[2], si2)
        pltpu.async_copy(idx.at[w, 3], ring.at[3], si3)
        pltpu.async_copy(table.at[ring.at[0, 0]], rows0, sem0)
        pltpu.async_copy(table.at[ring.at[1, 0]], rows1, sem1)
        pltpu.sync_copy(zeros, acc.at[pl.ds(s * RPT, RPT)])
        if with_counts:
            pltpu.sync_copy(ones, ones_v)
            pltpu.sync_copy(zeros_c, acc_c.at[pl.ds(s * RPT, RPT)])
        plsc.subcore_barrier()

        def body(i, carry):
            j = i * 4
            for p in range(4):
                b = j + p
                rbuf = rows0 if p % 2 == 0 else rows1
                rsem = sem0 if p % 2 == 0 else sem1
                gslot = (p + 2) % 4
                pltpu.make_async_copy(
                    table.at[ring.at[p, 0]], rbuf, rsem).wait()
                if with_counts:
                    pltpu.async_copy(
                        ones_v, acc_c.at[ring.at[p, 1]], sem_o, add=True)
                pltpu.sync_copy(rbuf, acc.at[ring.at[p, 1]], add=True)
                if with_counts:
                    pltpu.make_async_copy(
                        ones_v, acc_c.at[ring.at[p, 1]], sem_o).wait()

                @pl.when(b + 4 < NBLK)
                def _():
                    pltpu.async_copy(idx.at[w, b + 4], ring.at[p], sis[p])

                @pl.when(b + 2 < NBLK)
                def _():
                    pltpu.make_async_copy(
                        idx.at[w, b + 2], ring.at[gslot], sis[gslot]).wait()
                    pltpu.async_copy(
                        table.at[ring.at[gslot, 0]], rbuf, rsem)

            return carry

        lax.fori_loop(0, NBLK // 4, body, 0)
        plsc.subcore_barrier()
        pltpu.sync_copy(acc.at[pl.ds(s * RPT, RPT)],
                        out.at[c, pl.ds(s * RPT, RPT)])
        if with_counts:
            pltpu.sync_copy(acc_c.at[pl.ds(s * RPT, RPT)],
                            out_c.at[c, pl.ds(s * RPT, RPT)])

    return _sc_pass


def _prep_body(x_ref, w0_ref, b0_ref, w1_ref, out_ref):
    h = jnp.maximum(
        jnp.dot(x_ref[...], w0_ref[...], preferred_element_type=jnp.float32)
        + b0_ref[...], 0.0)
    out_ref[...] = jnp.dot(h, w1_ref[...], preferred_element_type=jnp.float32)


def _safe_inv(v):
    return jnp.where(v > 0, 1.0 / jnp.where(v > 0, v, 1.0), 0.0)


def _combine_body(p_ref, pc_ref, out_ref):
    p = p_ref[0] + p_ref[1]
    cnt = (pc_ref[0] + pc_ref[1])[:, 0:1]
    out_ref[...] = p * _safe_inv(cnt)


def _post1_body(p_ref, pc_ref, bc_ref, w2_ref, batch_ref,
                y2_ref, pool_ref, cnt_ref):
    i = pl.program_id(0)
    p = p_ref[0] + p_ref[1]
    d = (pc_ref[0] + pc_ref[1])[:, 0:1]
    h = jnp.maximum(p * _safe_inv(d) + bc_ref[...], 0.0)
    y2_ref[...] = jnp.dot(h, w2_ref[...], preferred_element_type=jnp.float32)
    b = batch_ref[0]
    gi = lax.broadcasted_iota(jnp.int32, (G, RB), 0)
    oh = (gi == b).astype(jnp.float32)

    @pl.when(i == 0)
    def _():
        pool_ref[...] = jnp.zeros_like(pool_ref)
        cnt_ref[...] = jnp.zeros_like(cnt_ref)

    pool_ref[...] += jnp.dot(oh, h, preferred_element_type=jnp.float32)
    cnt_ref[...] += jnp.sum(oh, axis=1, keepdims=True)


def _post2_body(p_ref, pc_ref, bc_ref, batch_ref, p1_ref, c_ref, wl_ref,
                bl_ref, w1_ref, b1_ref, g1_ref, be1_ref, w2_ref, b2_ref,
                g2_ref, be2_ref, wo_ref, pool_ref, out_ref):
    i = pl.program_id(0)
    p = p_ref[0] + p_ref[1]
    d = (pc_ref[0] + pc_ref[1])[:, 0:1]
    h = jnp.maximum(p * _safe_inv(d) + bc_ref[...], 0.0)
    b = batch_ref[0]
    gi = lax.broadcasted_iota(jnp.int32, (G, RB), 0)
    oh = (gi == b).astype(jnp.float32)

    @pl.when(i == 0)
    def _():
        pool_ref[...] = jnp.zeros_like(pool_ref)

    pool_ref[...] += jnp.dot(oh, h, preferred_element_type=jnp.float32)

    @pl.when(i == NRB - 1)
    def _():
        cnt = jnp.maximum(c_ref[...], 1.0)
        p1 = p1_ref[...] / cnt
        p2 = pool_ref[...] / cnt
        glob = jnp.concatenate([p1, p2], axis=1)
        he = (jnp.dot(glob, wl_ref[...], preferred_element_type=jnp.float32)
              + bl_ref[...])
        s = 0.9999950000374997
        m = jnp.maximum(
            (jnp.dot(he, w1_ref[...], preferred_element_type=jnp.float32)
             + b1_ref[...]) * s * g1_ref[...] + be1_ref[...], 0.0)
        m = jnp.maximum(
            (jnp.dot(m, w2_ref[...], preferred_element_type=jnp.float32)
             + b2_ref[...]) * s * g2_ref[...] + be2_ref[...], 0.0)
        out_ref[...] = jnp.dot(m, wo_ref[...], preferred_element_type=jnp.float32)


def _full(shape):
    return pl.BlockSpec(shape, lambda i: tuple(0 for _ in shape))


_prep = pl.pallas_call(
    _prep_body,
    grid=(NRB,),
    in_specs=[
        pl.BlockSpec((RB, D), lambda i: (i, 0)),
        _full((D, D)),
        _full((1, D)),
        _full((D, D)),
    ],
    out_specs=pl.BlockSpec((RB, D), lambda i: (i, 0)),
    out_shape=jax.ShapeDtypeStruct((N_NODES, D), jnp.float32),
)

_combine = pl.pallas_call(
    _combine_body,
    grid=(NRB,),
    in_specs=[
        pl.BlockSpec((NC, RB, D), lambda i: (0, i, 0)),
        pl.BlockSpec((NC, RB, CW), lambda i: (0, i, 0)),
    ],
    out_specs=pl.BlockSpec((RB, D), lambda i: (i, 0)),
    out_shape=jax.ShapeDtypeStruct((N_NODES, D), jnp.float32),
)

_post1 = pl.pallas_call(
    _post1_body,
    grid=(NRB,),
    in_specs=[
        pl.BlockSpec((NC, RB, D), lambda i: (0, i, 0)),
        pl.BlockSpec((NC, RB, CW), lambda i: (0, i, 0)),
        _full((1, D)),
        _full((D, D)),
        pl.BlockSpec((1, 1, RB), lambda i: (i, 0, 0)),
    ],
    out_specs=(
        pl.BlockSpec((RB, D), lambda i: (i, 0)),
        pl.BlockSpec((G, D), lambda i: (0, 0)),
        pl.BlockSpec((G, 1), lambda i: (0, 0)),
    ),
    out_shape=(
        jax.ShapeDtypeStruct((N_NODES, D), jnp.float32),
        jax.ShapeDtypeStruct((G, D), jnp.float32),
        jax.ShapeDtypeStruct((G, 1), jnp.float32),
    ),
)

_post2 = pl.pallas_call(
    _post2_body,
    grid=(NRB,),
    in_specs=[
        pl.BlockSpec((NC, RB, D), lambda i: (0, i, 0)),
        pl.BlockSpec((NC, RB, CW), lambda i: (0, i, 0)),
        _full((1, D)),
        pl.BlockSpec((1, 1, RB), lambda i: (i, 0, 0)),
        _full((G, D)),
        _full((G, 1)),
        _full((256, D)),
        _full((1, D)),
        _full((D, 64)),
        _full((1, 64)),
        _full((1, 64)),
        _full((1, 64)),
        _full((64, 32)),
        _full((1, 32)),
        _full((1, 32)),
        _full((1, 32)),
        _full((32, 4)),
    ],
    out_specs=(
        pl.BlockSpec((G, D), lambda i: (0, 0)),
        pl.BlockSpec((G, 4), lambda i: (0, 0)),
    ),
    out_shape=(
        jax.ShapeDtypeStruct((G, D), jnp.float32),
        jax.ShapeDtypeStruct((G, 4), jnp.float32),
    ),
)


def kernel(x, edge_index, batch, W_first, b_first, Wc1, bc1, Wc2, bc2, W_lin,
           b_lin, Wm1, bm1, g1, be1, Wm2, bm2, g2, be2, Wout):
    f32 = jnp.float32
    node_idx = edge_index[0]
    hedge_idx = edge_index[1]

    npad = NNZ_P - NNZ
    pad_g = (jnp.arange(npad, dtype=jnp.int32) * 41) % N_NODES
    pad_s = N_HEDGES + jnp.arange(npad, dtype=jnp.int32) % (NP - N_HEDGES)
    gidx1 = jnp.concatenate([node_idx, pad_g]).reshape(NW, NBLK, K)
    sidx1 = jnp.concatenate([hedge_idx, pad_s]).reshape(NW, NBLK, K)
    gidx2 = jnp.concatenate([hedge_idx, pad_g]).reshape(NW, NBLK, K)
    sidx2 = jnp.concatenate([node_idx, pad_s]).reshape(NW, NBLK, K)
    idx1 = jnp.stack([gidx1, sidx1], axis=2)
    idx2 = jnp.stack([gidx2, sidx2], axis=2)

    zeros = jnp.zeros((RPT, D), f32)
    zeros_c = jnp.zeros((RPT, CW), f32)
    ones = jnp.ones((K, CW), f32)
    batch_r = batch.reshape(NRB, 1, RB)

    sc_c = _make_sc_pass(True)
    sc_n = _make_sc_pass(False)
    y1 = _prep(x, W_first, b_first.reshape(1, D), Wc1)
    pm1, cb = sc_c(y1, idx1, zeros, zeros_c, ones)
    m1 = _combine(pm1, cb)
    po1, cd = sc_c(m1, idx2, zeros, zeros_c, ones)
    y2, pool1, cntg = _post1(po1, cd, bc1.reshape(1, D), Wc2, batch_r)
    pm2 = sc_n(y2, idx1, zeros, zeros_c, ones)
    m2 = _combine(pm2, cb)
    po2 = sc_n(m2, idx2, zeros, zeros_c, ones)
    _pool2, out = _post2(po2, cd, bc2.reshape(1, D), batch_r, pool1, cntg,
                         W_lin, b_lin.reshape(1, D), Wm1, bm1.reshape(1, 64),
                         g1.reshape(1, 64), be1.reshape(1, 64), Wm2,
                         bm2.reshape(1, 32), g2.reshape(1, 32),
                         be2.reshape(1, 32), Wout)
    return out


# speedup vs baseline: 1.1680x; 1.0754x over previous
import functools

import jax
import jax.numpy as jnp
from jax import lax
from jax.experimental import pallas as pl
from jax.experimental.pallas import tpu as pltpu
from jax.experimental.pallas import tpu_sc as plsc

N_NODES = 10000
N_HEDGES = 10000
NNZ = 320000
D = 128
G = 8
CW = 16

NC, NS = 2, 16
NW = NC * NS
K = 128
NP = 10240
EPT = NP
NNZ_P = NW * EPT
NBLK = EPT // K
RPT = NP // NS

RB = 1000
NRB = N_NODES // RB


@functools.cache
def _make_sc_pass(with_counts):
    mesh = plsc.VectorSubcoreMesh(
        core_axis_name="c", subcore_axis_name="s", num_cores=NC, num_subcores=NS)

    if with_counts:
        out_type = (
            jax.ShapeDtypeStruct((NC, NP, D), jnp.float32),
            jax.ShapeDtypeStruct((NC, NP, CW), jnp.float32),
        )
        extra = [
            pltpu.VMEM((K, CW), jnp.float32),
            pltpu.VMEM_SHARED((NP, CW), jnp.float32),
            pltpu.SemaphoreType.DMA,
        ]
    else:
        out_type = jax.ShapeDtypeStruct((NC, NP, D), jnp.float32)
        extra = []

    @functools.partial(
        pl.kernel,
        out_type=out_type,
        mesh=mesh,
        scratch_types=[
            pltpu.VMEM((4, 2, K), jnp.int32),
            pltpu.VMEM((K, D), jnp.float32),
            pltpu.VMEM((K, D), jnp.float32),
        ] + extra + [
            pltpu.VMEM_SHARED((NP, D), jnp.float32),
            pltpu.SemaphoreType.DMA,
            pltpu.SemaphoreType.DMA,
            pltpu.SemaphoreType.DMA,
            pltpu.SemaphoreType.DMA,
            pltpu.SemaphoreType.DMA,
            pltpu.SemaphoreType.DMA,
        ],
        compiler_params=pltpu.CompilerParams(use_tc_tiling_on_sc=False),
    )
    def _sc_pass(table, idx, zeros, zeros_c, ones, *rest):
        if with_counts:
            (out, out_c, ring, rows0, rows1, ones_v, acc_c, sem_o, acc,
             sem0, sem1, si0, si1, si2, si3) = rest
        else:
            (out, ring, rows0, rows1, acc,
             sem0, sem1, si0, si1, si2, si3) = rest
        c = lax.axis_index("c")
        s = lax.axis_index("s")
        w = c * NS + s
        sis = (si0, si1, si2, si3)

        pltpu.sync_copy(idx.at[w, 0], ring.at[0])
        pltpu.sync_copy(idx.at[w, 1], ring.at[1])
        pltpu.async_copy(idx.at[w, 2], ring.at[2], si2)
        pltpu.async_copy(idx.at[w, 3], ring.at[3], si3)
        pltpu.async_copy(table.at[ring.at[0, 0]], rows0, sem0)
        pltpu.async_copy(table.at[ring.at[1, 0]], rows1, sem1)
        pltpu.sync_copy(zeros, acc.at[pl.ds(s * RPT, RPT)])
        if with_counts:
            pltpu.sync_copy(ones, ones_v)
            pltpu.sync_copy(zeros_c, acc_c.at[pl.ds(s * RPT, RPT)])
        plsc.subcore_barrier()

        def body(i, carry):
            j = i * 4
            for p in range(4):
                b = j + p
                rbuf = rows0 if p % 2 == 0 else rows1
                rsem = sem0 if p % 2 == 0 else sem1
                gslot = (p + 2) % 4
                pltpu.make_async_copy(
                    table.at[ring.at[p, 0]], rbuf, rsem).wait()
                if with_counts:
                    pltpu.async_copy(
                        ones_v, acc_c.at[ring.at[p, 1]], sem_o, add=True)
                pltpu.sync_copy(rbuf, acc.at[ring.at[p, 1]], add=True)
                if with_counts:
                    pltpu.make_async_copy(
                        ones_v, acc_c.at[ring.at[p, 1]], sem_o).wait()

                @pl.when(b + 4 < NBLK)
                def _():
                    pltpu.async_copy(idx.at[w, b + 4], ring.at[p], sis[p])

                @pl.when(b + 2 < NBLK)
                def _():
                    pltpu.make_async_copy(
                        idx.at[w, b + 2], ring.at[gslot], sis[gslot]).wait()
                    pltpu.async_copy(
                        table.at[ring.at[gslot, 0]], rbuf, rsem)

            return carry

        lax.fori_loop(0, NBLK // 4, body, 0)
        plsc.subcore_barrier()
        pltpu.sync_copy(acc.at[pl.ds(s * RPT, RPT)],
                        out.at[c, pl.ds(s * RPT, RPT)])
        if with_counts:
            pltpu.sync_copy(acc_c.at[pl.ds(s * RPT, RPT)],
                            out_c.at[c, pl.ds(s * RPT, RPT)])

    return _sc_pass


def _prep_body(x_ref, w0_ref, b0_ref, w1_ref, out_ref):
    h = jnp.maximum(
        jnp.dot(x_ref[...], w0_ref[...], preferred_element_type=jnp.float32)
        + b0_ref[...], 0.0)
    out_ref[...] = jnp.dot(h, w1_ref[...], preferred_element_type=jnp.float32)


def _safe_inv(v):
    return jnp.where(v > 0, 1.0 / jnp.where(v > 0, v, 1.0), 0.0)


def _combine_body(p_ref, pc_ref, out_ref):
    p = p_ref[0] + p_ref[1]
    cnt = (pc_ref[0] + pc_ref[1])[:, 0:1]
    out_ref[...] = p * _safe_inv(cnt)


def _post1_body(p_ref, pc_ref, bc_ref, w2_ref, batch_ref,
                y2_ref, pool_ref, cnt_ref):
    i = pl.program_id(0)
    p = p_ref[0] + p_ref[1]
    d = (pc_ref[0] + pc_ref[1])[:, 0:1]
    h = jnp.maximum(p * _safe_inv(d) + bc_ref[...], 0.0)
    y2_ref[...] = jnp.dot(h, w2_ref[...], preferred_element_type=jnp.float32)
    b = batch_ref[0]
    gi = lax.broadcasted_iota(jnp.int32, (G, RB), 0)
    oh = (gi == b).astype(jnp.float32)

    @pl.when(i == 0)
    def _():
        pool_ref[...] = jnp.zeros_like(pool_ref)
        cnt_ref[...] = jnp.zeros_like(cnt_ref)

    pool_ref[...] += jnp.dot(oh, h, preferred_element_type=jnp.float32)
    cnt_ref[...] += jnp.sum(oh, axis=1, keepdims=True)


def _post2_body(p_ref, pc_ref, bc_ref, batch_ref, p1_ref, c_ref, wl_ref,
                bl_ref, w1_ref, b1_ref, g1_ref, be1_ref, w2_ref, b2_ref,
                g2_ref, be2_ref, wo_ref, pool_ref, out_ref):
    i = pl.program_id(0)
    p = p_ref[0] + p_ref[1]
    d = (pc_ref[0] + pc_ref[1])[:, 0:1]
    h = jnp.maximum(p * _safe_inv(d) + bc_ref[...], 0.0)
    b = batch_ref[0]
    gi = lax.broadcasted_iota(jnp.int32, (G, RB), 0)
    oh = (gi == b).astype(jnp.float32)

    @pl.when(i == 0)
    def _():
        pool_ref[...] = jnp.zeros_like(pool_ref)

    pool_ref[...] += jnp.dot(oh, h, preferred_element_type=jnp.float32)

    @pl.when(i == NRB - 1)
    def _():
        cnt = jnp.maximum(c_ref[...], 1.0)
        p1 = p1_ref[...] / cnt
        p2 = pool_ref[...] / cnt
        glob = jnp.concatenate([p1, p2], axis=1)
        he = (jnp.dot(glob, wl_ref[...], preferred_element_type=jnp.float32)
              + bl_ref[...])
        s = 0.9999950000374997
        m = jnp.maximum(
            (jnp.dot(he, w1_ref[...], preferred_element_type=jnp.float32)
             + b1_ref[...]) * s * g1_ref[...] + be1_ref[...], 0.0)
        m = jnp.maximum(
            (jnp.dot(m, w2_ref[...], preferred_element_type=jnp.float32)
             + b2_ref[...]) * s * g2_ref[...] + be2_ref[...], 0.0)
        out_ref[...] = jnp.dot(m, wo_ref[...], preferred_element_type=jnp.float32)


def _full(shape):
    return pl.BlockSpec(shape, lambda i: tuple(0 for _ in shape))


_prep = pl.pallas_call(
    _prep_body,
    grid=(NRB,),
    in_specs=[
        pl.BlockSpec((RB, D), lambda i: (i, 0)),
        _full((D, D)),
        _full((1, D)),
        _full((D, D)),
    ],
    out_specs=pl.BlockSpec((RB, D), lambda i: (i, 0)),
    out_shape=jax.ShapeDtypeStruct((N_NODES, D), jnp.float32),
)

_combine = pl.pallas_call(
    _combine_body,
    grid=(NRB,),
    in_specs=[
        pl.BlockSpec((NC, RB, D), lambda i: (0, i, 0)),
        pl.BlockSpec((NC, RB, CW), lambda i: (0, i, 0)),
    ],
    out_specs=pl.BlockSpec((RB, D), lambda i: (i, 0)),
    out_shape=jax.ShapeDtypeStruct((N_NODES, D), jnp.float32),
)

_post1 = pl.pallas_call(
    _post1_body,
    grid=(NRB,),
    in_specs=[
        pl.BlockSpec((NC, RB, D), lambda i: (0, i, 0)),
        pl.BlockSpec((NC, RB, CW), lambda i: (0, i, 0)),
        _full((1, D)),
        _full((D, D)),
        pl.BlockSpec((1, 1, RB), lambda i: (i, 0, 0)),
    ],
    out_specs=(
        pl.BlockSpec((RB, D), lambda i: (i, 0)),
        pl.BlockSpec((G, D), lambda i: (0, 0)),
        pl.BlockSpec((G, 1), lambda i: (0, 0)),
    ),
    out_shape=(
        jax.ShapeDtypeStruct((N_NODES, D), jnp.float32),
        jax.ShapeDtypeStruct((G, D), jnp.float32),
        jax.ShapeDtypeStruct((G, 1), jnp.float32),
    ),
)

_post2 = pl.pallas_call(
    _post2_body,
    grid=(NRB,),
    in_specs=[
        pl.BlockSpec((NC, RB, D), lambda i: (0, i, 0)),
        pl.BlockSpec((NC, RB, CW), lambda i: (0, i, 0)),
        _full((1, D)),
        pl.BlockSpec((1, 1, RB), lambda i: (i, 0, 0)),
        _full((G, D)),
        _full((G, 1)),
        _full((256, D)),
        _full((1, D)),
        _full((D, 64)),
        _full((1, 64)),
        _full((1, 64)),
        _full((1, 64)),
        _full((64, 32)),
        _full((1, 32)),
        _full((1, 32)),
        _full((1, 32)),
        _full((32, 4)),
    ],
    out_specs=(
        pl.BlockSpec((G, D), lambda i: (0, 0)),
        pl.BlockSpec((G, 4), lambda i: (0, 0)),
    ),
    out_shape=(
        jax.ShapeDtypeStruct((G, D), jnp.float32),
        jax.ShapeDtypeStruct((G, 4), jnp.float32),
    ),
)


def kernel(x, edge_index, batch, W_first, b_first, Wc1, bc1, Wc2, bc2, W_lin,
           b_lin, Wm1, bm1, g1, be1, Wm2, bm2, g2, be2, Wout):
    f32 = jnp.float32
    node_idx = edge_index[0]
    hedge_idx = edge_index[1]

    npad = NNZ_P - NNZ
    pad_g = (jnp.arange(npad, dtype=jnp.int32) * 41) % N_NODES
    pad_s = N_HEDGES + jnp.arange(npad, dtype=jnp.int32) % (NP - N_HEDGES)
    gidx1 = jnp.concatenate([node_idx, pad_g]).reshape(NW, NBLK, K)
    sidx1 = jnp.concatenate([hedge_idx, pad_s]).reshape(NW, NBLK, K)
    gidx2 = jnp.concatenate([hedge_idx, pad_g]).reshape(NW, NBLK, K)
    sidx2 = jnp.concatenate([node_idx, pad_s]).reshape(NW, NBLK, K)
    idx1 = jnp.stack([gidx1, sidx1], axis=2)
    idx2 = jnp.stack([gidx2, sidx2], axis=2)

    zeros = jnp.zeros((RPT, D), f32)
    zeros_c = jnp.zeros((RPT, CW), f32)
    ones = jnp.ones((K, CW), f32)
    batch_r = batch.reshape(NRB, 1, RB)

    sc_c = _make_sc_pass(True)
    sc_n = _make_sc_pass(False)
    y1 = _prep(x, W_first, b_first.reshape(1, D), Wc1)
    pm1, cb = sc_c(y1, idx1, zeros, zeros_c, ones)
    m1 = _combine(pm1, cb)
    po1, cd = sc_c(m1, idx2, zeros, zeros_c, ones)
    y2, pool1, cntg = _post1(po1, cd, bc1.reshape(1, D), Wc2, batch_r)
    pm2 = sc_n(y2, idx1, zeros, zeros_c, ones)
    m2 = _combine(pm2, cb)
    po2 = sc_n(m2, idx2, zeros, zeros_c, ones)
    _pool2, out = _post2(po2, cd, bc2.reshape(1, D), batch_r, pool1, cntg,
                         W_lin, b_lin.reshape(1, D), Wm1, bm1.reshape(1, 64),
                         g1.reshape(1, 64), be1.reshape(1, 64), Wm2,
                         bm2.reshape(1, 32), g2.reshape(1, 32),
                         be2.reshape(1, 32), Wout)
    return out


# speedup vs baseline: 1.1921x; 1.0206x over previous
import functools

import jax
import jax.numpy as jnp
from jax import lax
from jax.experimental import pallas as pl
from jax.experimental.pallas import tpu as pltpu
from jax.experimental.pallas import tpu_sc as plsc

N_NODES = 10000
N_HEDGES = 10000
NNZ = 320000
D = 128
G = 8
CW = 16

NC, NS = 2, 16
NW = NC * NS
K = 128
NP = 10240
EPT = NP
NNZ_P = NW * EPT
NBLK = EPT // K
RPT = NP // NS

RB = 2000
NRB = N_NODES // RB


@functools.cache
def _make_sc_pass(with_counts):
    mesh = plsc.VectorSubcoreMesh(
        core_axis_name="c", subcore_axis_name="s", num_cores=NC, num_subcores=NS)

    if with_counts:
        out_type = (
            jax.ShapeDtypeStruct((NC, NP, D), jnp.float32),
            jax.ShapeDtypeStruct((NC, NP, CW), jnp.float32),
        )
        extra = [
            pltpu.VMEM((K, CW), jnp.float32),
            pltpu.VMEM_SHARED((NP, CW), jnp.float32),
            pltpu.SemaphoreType.DMA,
        ]
    else:
        out_type = jax.ShapeDtypeStruct((NC, NP, D), jnp.float32)
        extra = []

    @functools.partial(
        pl.kernel,
        out_type=out_type,
        mesh=mesh,
        scratch_types=[
            pltpu.VMEM((4, 2, K), jnp.int32),
            pltpu.VMEM((K, D), jnp.float32),
            pltpu.VMEM((K, D), jnp.float32),
        ] + extra + [
            pltpu.VMEM_SHARED((NP, D), jnp.float32),
            pltpu.SemaphoreType.DMA,
            pltpu.SemaphoreType.DMA,
            pltpu.SemaphoreType.DMA,
            pltpu.SemaphoreType.DMA,
            pltpu.SemaphoreType.DMA,
            pltpu.SemaphoreType.DMA,
        ],
        compiler_params=pltpu.CompilerParams(use_tc_tiling_on_sc=False),
    )
    def _sc_pass(table, idx, zeros, zeros_c, ones, *rest):
        if with_counts:
            (out, out_c, ring, rows0, rows1, ones_v, acc_c, sem_o, acc,
             sem0, sem1, si0, si1, si2, si3) = rest
        else:
            (out, ring, rows0, rows1, acc,
             sem0, sem1, si0, si1, si2, si3) = rest
        c = lax.axis_index("c")
        s = lax.axis_index("s")
        w = c * NS + s
        sis = (si0, si1, si2, si3)

        pltpu.sync_copy(idx.at[w, 0], ring.at[0])
        pltpu.sync_copy(idx.at[w, 1], ring.at[1])
        pltpu.async_copy(idx.at[w, 2], ring.at[2], si2)
        pltpu.async_copy(idx.at[w, 3], ring.at[3], si3)
        pltpu.async_copy(table.at[ring.at[0, 0]], rows0, sem0)
        pltpu.async_copy(table.at[ring.at[1, 0]], rows1, sem1)
        pltpu.sync_copy(zeros, acc.at[pl.ds(s * RPT, RPT)])
        if with_counts:
            pltpu.sync_copy(ones, ones_v)
            pltpu.sync_copy(zeros_c, acc_c.at[pl.ds(s * RPT, RPT)])
        plsc.subcore_barrier()

        def body(i, carry):
            j = i * 4
            for p in range(4):
                b = j + p
                rbuf = rows0 if p % 2 == 0 else rows1
                rsem = sem0 if p % 2 == 0 else sem1
                gslot = (p + 2) % 4
                pltpu.make_async_copy(
                    table.at[ring.at[p, 0]], rbuf, rsem).wait()
                if with_counts:
                    pltpu.async_copy(
                        ones_v, acc_c.at[ring.at[p, 1]], sem_o, add=True)
                pltpu.sync_copy(rbuf, acc.at[ring.at[p, 1]], add=True)
                if with_counts:
                    pltpu.make_async_copy(
                        ones_v, acc_c.at[ring.at[p, 1]], sem_o).wait()

                @pl.when(b + 4 < NBLK)
                def _():
                    pltpu.async_copy(idx.at[w, b + 4], ring.at[p], sis[p])

                @pl.when(b + 2 < NBLK)
                def _():
                    pltpu.make_async_copy(
                        idx.at[w, b + 2], ring.at[gslot], sis[gslot]).wait()
                    pltpu.async_copy(
                        table.at[ring.at[gslot, 0]], rbuf, rsem)

            return carry

        lax.fori_loop(0, NBLK // 4, body, 0)
        plsc.subcore_barrier()
        pltpu.sync_copy(acc.at[pl.ds(s * RPT, RPT)],
                        out.at[c, pl.ds(s * RPT, RPT)])
        if with_counts:
            pltpu.sync_copy(acc_c.at[pl.ds(s * RPT, RPT)],
                            out_c.at[c, pl.ds(s * RPT, RPT)])

    return _sc_pass


def _prep_body(x_ref, w0_ref, b0_ref, w1_ref, out_ref):
    h = jnp.maximum(
        jnp.dot(x_ref[...], w0_ref[...], preferred_element_type=jnp.float32)
        + b0_ref[...], 0.0)
    out_ref[...] = jnp.dot(h, w1_ref[...], preferred_element_type=jnp.float32)


def _safe_inv(v):
    return jnp.where(v > 0, 1.0 / jnp.where(v > 0, v, 1.0), 0.0)


def _combine_body(p_ref, pc_ref, out_ref):
    p = p_ref[0] + p_ref[1]
    cnt = (pc_ref[0] + pc_ref[1])[:, 0:1]
    out_ref[...] = p * _safe_inv(cnt)


def _post1_body(p_ref, pc_ref, bc_ref, w2_ref, batch_ref,
                y2_ref, pool_ref, cnt_ref):
    i = pl.program_id(0)
    p = p_ref[0] + p_ref[1]
    d = (pc_ref[0] + pc_ref[1])[:, 0:1]
    h = jnp.maximum(p * _safe_inv(d) + bc_ref[...], 0.0)
    y2_ref[...] = jnp.dot(h, w2_ref[...], preferred_element_type=jnp.float32)
    b = batch_ref[0]
    gi = lax.broadcasted_iota(jnp.int32, (G, RB), 0)
    oh = (gi == b).astype(jnp.float32)

    @pl.when(i == 0)
    def _():
        pool_ref[...] = jnp.zeros_like(pool_ref)
        cnt_ref[...] = jnp.zeros_like(cnt_ref)

    pool_ref[...] += jnp.dot(oh, h, preferred_element_type=jnp.float32)
    cnt_ref[...] += jnp.sum(oh, axis=1, keepdims=True)


def _post2_body(p_ref, pc_ref, bc_ref, batch_ref, p1_ref, c_ref, wl_ref,
                bl_ref, w1_ref, b1_ref, g1_ref, be1_ref, w2_ref, b2_ref,
                g2_ref, be2_ref, wo_ref, pool_ref, out_ref):
    i = pl.program_id(0)
    p = p_ref[0] + p_ref[1]
    d = (pc_ref[0] + pc_ref[1])[:, 0:1]
    h = jnp.maximum(p * _safe_inv(d) + bc_ref[...], 0.0)
    b = batch_ref[0]
    gi = lax.broadcasted_iota(jnp.int32, (G, RB), 0)
    oh = (gi == b).astype(jnp.float32)

    @pl.when(i == 0)
    def _():
        pool_ref[...] = jnp.zeros_like(pool_ref)

    pool_ref[...] += jnp.dot(oh, h, preferred_element_type=jnp.float32)

    @pl.when(i == NRB - 1)
    def _():
        cnt = jnp.maximum(c_ref[...], 1.0)
        p1 = p1_ref[...] / cnt
        p2 = pool_ref[...] / cnt
        glob = jnp.concatenate([p1, p2], axis=1)
        he = (jnp.dot(glob, wl_ref[...], preferred_element_type=jnp.float32)
              + bl_ref[...])
        s = 0.9999950000374997
        m = jnp.maximum(
            (jnp.dot(he, w1_ref[...], preferred_element_type=jnp.float32)
             + b1_ref[...]) * s * g1_ref[...] + be1_ref[...], 0.0)
        m = jnp.maximum(
            (jnp.dot(m, w2_ref[...], preferred_element_type=jnp.float32)
             + b2_ref[...]) * s * g2_ref[...] + be2_ref[...], 0.0)
        out_ref[...] = jnp.dot(m, wo_ref[...], preferred_element_type=jnp.float32)


def _full(shape):
    return pl.BlockSpec(shape, lambda i: tuple(0 for _ in shape))


_prep = pl.pallas_call(
    _prep_body,
    grid=(NRB,),
    in_specs=[
        pl.BlockSpec((RB, D), lambda i: (i, 0)),
        _full((D, D)),
        _full((1, D)),
        _full((D, D)),
    ],
    out_specs=pl.BlockSpec((RB, D), lambda i: (i, 0)),
    out_shape=jax.ShapeDtypeStruct((N_NODES, D), jnp.float32),
)

_combine = pl.pallas_call(
    _combine_body,
    grid=(NRB,),
    in_specs=[
        pl.BlockSpec((NC, RB, D), lambda i: (0, i, 0)),
        pl.BlockSpec((NC, RB, CW), lambda i: (0, i, 0)),
    ],
    out_specs=pl.BlockSpec((RB, D), lambda i: (i, 0)),
    out_shape=jax.ShapeDtypeStruct((N_NODES, D), jnp.float32),
)

_post1 = pl.pallas_call(
    _post1_body,
    grid=(NRB,),
    in_specs=[
        pl.BlockSpec((NC, RB, D), lambda i: (0, i, 0)),
        pl.BlockSpec((NC, RB, CW), lambda i: (0, i, 0)),
        _full((1, D)),
        _full((D, D)),
        pl.BlockSpec((1, 1, RB), lambda i: (i, 0, 0)),
    ],
    out_specs=(
        pl.BlockSpec((RB, D), lambda i: (i, 0)),
        pl.BlockSpec((G, D), lambda i: (0, 0)),
        pl.BlockSpec((G, 1), lambda i: (0, 0)),
    ),
    out_shape=(
        jax.ShapeDtypeStruct((N_NODES, D), jnp.float32),
        jax.ShapeDtypeStruct((G, D), jnp.float32),
        jax.ShapeDtypeStruct((G, 1), jnp.float32),
    ),
)

_post2 = pl.pallas_call(
    _post2_body,
    grid=(NRB,),
    in_specs=[
        pl.BlockSpec((NC, RB, D), lambda i: (0, i, 0)),
        pl.BlockSpec((NC, RB, CW), lambda i: (0, i, 0)),
        _full((1, D)),
        pl.BlockSpec((1, 1, RB), lambda i: (i, 0, 0)),
        _full((G, D)),
        _full((G, 1)),
        _full((256, D)),
        _full((1, D)),
        _full((D, 64)),
        _full((1, 64)),
        _full((1, 64)),
        _full((1, 64)),
        _full((64, 32)),
        _full((1, 32)),
        _full((1, 32)),
        _full((1, 32)),
        _full((32, 4)),
    ],
    out_specs=(
        pl.BlockSpec((G, D), lambda i: (0, 0)),
        pl.BlockSpec((G, 4), lambda i: (0, 0)),
    ),
    out_shape=(
        jax.ShapeDtypeStruct((G, D), jnp.float32),
        jax.ShapeDtypeStruct((G, 4), jnp.float32),
    ),
)


def kernel(x, edge_index, batch, W_first, b_first, Wc1, bc1, Wc2, bc2, W_lin,
           b_lin, Wm1, bm1, g1, be1, Wm2, bm2, g2, be2, Wout):
    f32 = jnp.float32
    node_idx = edge_index[0]
    hedge_idx = edge_index[1]

    npad = NNZ_P - NNZ
    pad_g = (jnp.arange(npad, dtype=jnp.int32) * 41) % N_NODES
    pad_s = N_HEDGES + jnp.arange(npad, dtype=jnp.int32) % (NP - N_HEDGES)
    gidx1 = jnp.concatenate([node_idx, pad_g]).reshape(NW, NBLK, K)
    sidx1 = jnp.concatenate([hedge_idx, pad_s]).reshape(NW, NBLK, K)
    gidx2 = jnp.concatenate([hedge_idx, pad_g]).reshape(NW, NBLK, K)
    sidx2 = jnp.concatenate([node_idx, pad_s]).reshape(NW, NBLK, K)
    idx1 = jnp.stack([gidx1, sidx1], axis=2)
    idx2 = jnp.stack([gidx2, sidx2], axis=2)

    zeros = jnp.zeros((RPT, D), f32)
    zeros_c = jnp.zeros((RPT, CW), f32)
    ones = jnp.ones((K, CW), f32)
    batch_r = batch.reshape(NRB, 1, RB)

    sc_c = _make_sc_pass(True)
    sc_n = _make_sc_pass(False)
    y1 = _prep(x, W_first, b_first.reshape(1, D), Wc1)
    pm1, cb = sc_c(y1, idx1, zeros, zeros_c, ones)
    m1 = _combine(pm1, cb)
    po1, cd = sc_c(m1, idx2, zeros, zeros_c, ones)
    y2, pool1, cntg = _post1(po1, cd, bc1.reshape(1, D), Wc2, batch_r)
    pm2 = sc_n(y2, idx1, zeros, zeros_c, ones)
    m2 = _combine(pm2, cb)
    po2 = sc_n(m2, idx2, zeros, zeros_c, ones)
    _pool2, out = _post2(po2, cd, bc2.reshape(1, D), batch_r, pool1, cntg,
                         W_lin, b_lin.reshape(1, D), Wm1, bm1.reshape(1, 64),
                         g1.reshape(1, 64), be1.reshape(1, 64), Wm2,
                         bm2.reshape(1, 32), g2.reshape(1, 32),
                         be2.reshape(1, 32), Wout)
    return out


# speedup vs baseline: 1.1963x; 1.0035x over previous
import functools

import jax
import jax.numpy as jnp
from jax import lax
from jax.experimental import pallas as pl
from jax.experimental.pallas import tpu as pltpu
from jax.experimental.pallas import tpu_sc as plsc

N_NODES = 10000
N_HEDGES = 10000
NNZ = 320000
D = 128
G = 8
CW = 16

NC, NS = 2, 16
NW = NC * NS
K = 128
NP = 10240
EPT = NP
NNZ_P = NW * EPT
NBLK = EPT // K
RPT = NP // NS

RB = 5000
NRB = N_NODES // RB


@functools.cache
def _make_sc_pass(with_counts):
    mesh = plsc.VectorSubcoreMesh(
        core_axis_name="c", subcore_axis_name="s", num_cores=NC, num_subcores=NS)

    if with_counts:
        out_type = (
            jax.ShapeDtypeStruct((NC, NP, D), jnp.float32),
            jax.ShapeDtypeStruct((NC, NP, CW), jnp.float32),
        )
        extra = [
            pltpu.VMEM((K, CW), jnp.float32),
            pltpu.VMEM_SHARED((NP, CW), jnp.float32),
            pltpu.SemaphoreType.DMA,
        ]
    else:
        out_type = jax.ShapeDtypeStruct((NC, NP, D), jnp.float32)
        extra = []

    @functools.partial(
        pl.kernel,
        out_type=out_type,
        mesh=mesh,
        scratch_types=[
            pltpu.VMEM((4, 2, K), jnp.int32),
            pltpu.VMEM((K, D), jnp.float32),
            pltpu.VMEM((K, D), jnp.float32),
        ] + extra + [
            pltpu.VMEM_SHARED((NP, D), jnp.float32),
            pltpu.SemaphoreType.DMA,
            pltpu.SemaphoreType.DMA,
            pltpu.SemaphoreType.DMA,
            pltpu.SemaphoreType.DMA,
            pltpu.SemaphoreType.DMA,
            pltpu.SemaphoreType.DMA,
        ],
        compiler_params=pltpu.CompilerParams(use_tc_tiling_on_sc=False),
    )
    def _sc_pass(table, idx, zeros, zeros_c, ones, *rest):
        if with_counts:
            (out, out_c, ring, rows0, rows1, ones_v, acc_c, sem_o, acc,
             sem0, sem1, si0, si1, si2, si3) = rest
        else:
            (out, ring, rows0, rows1, acc,
             sem0, sem1, si0, si1, si2, si3) = rest
        c = lax.axis_index("c")
        s = lax.axis_index("s")
        w = c * NS + s
        sis = (si0, si1, si2, si3)

        pltpu.sync_copy(idx.at[w, 0], ring.at[0])
        pltpu.sync_copy(idx.at[w, 1], ring.at[1])
        pltpu.async_copy(idx.at[w, 2], ring.at[2], si2)
        pltpu.async_copy(idx.at[w, 3], ring.at[3], si3)
        pltpu.async_copy(table.at[ring.at[0, 0]], rows0, sem0)
        pltpu.async_copy(table.at[ring.at[1, 0]], rows1, sem1)
        pltpu.sync_copy(zeros, acc.at[pl.ds(s * RPT, RPT)])
        if with_counts:
            pltpu.sync_copy(ones, ones_v)
            pltpu.sync_copy(zeros_c, acc_c.at[pl.ds(s * RPT, RPT)])
        plsc.subcore_barrier()

        def body(i, carry):
            j = i * 4
            for p in range(4):
                b = j + p
                rbuf = rows0 if p % 2 == 0 else rows1
                rsem = sem0 if p % 2 == 0 else sem1
                gslot = (p + 2) % 4
                pltpu.make_async_copy(
                    table.at[ring.at[p, 0]], rbuf, rsem).wait()
                if with_counts:
                    pltpu.async_copy(
                        ones_v, acc_c.at[ring.at[p, 1]], sem_o, add=True)
                pltpu.sync_copy(rbuf, acc.at[ring.at[p, 1]], add=True)
                if with_counts:
                    pltpu.make_async_copy(
                        ones_v, acc_c.at[ring.at[p, 1]], sem_o).wait()

                @pl.when(b + 4 < NBLK)
                def _():
                    pltpu.async_copy(idx.at[w, b + 4], ring.at[p], sis[p])

                @pl.when(b + 2 < NBLK)
                def _():
                    pltpu.make_async_copy(
                        idx.at[w, b + 2], ring.at[gslot], sis[gslot]).wait()
                    pltpu.async_copy(
                        table.at[ring.at[gslot, 0]], rbuf, rsem)

            return carry

        lax.fori_loop(0, NBLK // 4, body, 0)
        plsc.subcore_barrier()
        pltpu.sync_copy(acc.at[pl.ds(s * RPT, RPT)],
                        out.at[c, pl.ds(s * RPT, RPT)])
        if with_counts:
            pltpu.sync_copy(acc_c.at[pl.ds(s * RPT, RPT)],
                            out_c.at[c, pl.ds(s * RPT, RPT)])

    return _sc_pass


def _prep_body(x_ref, w0_ref, b0_ref, w1_ref, out_ref):
    h = jnp.maximum(
        jnp.dot(x_ref[...], w0_ref[...], preferred_element_type=jnp.float32)
        + b0_ref[...], 0.0)
    out_ref[...] = jnp.dot(h, w1_ref[...], preferred_element_type=jnp.float32)


def _safe_inv(v):
    return jnp.where(v > 0, 1.0 / jnp.where(v > 0, v, 1.0), 0.0)


def _combine_body(p_ref, pc_ref, out_ref):
    p = p_ref[0] + p_ref[1]
    cnt = (pc_ref[0] + pc_ref[1])[:, 0:1]
    out_ref[...] = p * _safe_inv(cnt)


def _post1_body(p_ref, pc_ref, bc_ref, w2_ref, batch_ref,
                y2_ref, pool_ref, cnt_ref):
    i = pl.program_id(0)
    p = p_ref[0] + p_ref[1]
    d = (pc_ref[0] + pc_ref[1])[:, 0:1]
    h = jnp.maximum(p * _safe_inv(d) + bc_ref[...], 0.0)
    y2_ref[...] = jnp.dot(h, w2_ref[...], preferred_element_type=jnp.float32)
    b = batch_ref[0]
    gi = lax.broadcasted_iota(jnp.int32, (G, RB), 0)
    oh = (gi == b).astype(jnp.float32)

    @pl.when(i == 0)
    def _():
        pool_ref[...] = jnp.zeros_like(pool_ref)
        cnt_ref[...] = jnp.zeros_like(cnt_ref)

    pool_ref[...] += jnp.dot(oh, h, preferred_element_type=jnp.float32)
    cnt_ref[...] += jnp.sum(oh, axis=1, keepdims=True)


def _post2_body(p_ref, pc_ref, bc_ref, batch_ref, p1_ref, c_ref, wl_ref,
                bl_ref, w1_ref, b1_ref, g1_ref, be1_ref, w2_ref, b2_ref,
                g2_ref, be2_ref, wo_ref, pool_ref, out_ref):
    i = pl.program_id(0)
    p = p_ref[0] + p_ref[1]
    d = (pc_ref[0] + pc_ref[1])[:, 0:1]
    h = jnp.maximum(p * _safe_inv(d) + bc_ref[...], 0.0)
    b = batch_ref[0]
    gi = lax.broadcasted_iota(jnp.int32, (G, RB), 0)
    oh = (gi == b).astype(jnp.float32)

    @pl.when(i == 0)
    def _():
        pool_ref[...] = jnp.zeros_like(pool_ref)

    pool_ref[...] += jnp.dot(oh, h, preferred_element_type=jnp.float32)

    @pl.when(i == NRB - 1)
    def _():
        cnt = jnp.maximum(c_ref[...], 1.0)
        p1 = p1_ref[...] / cnt
        p2 = pool_ref[...] / cnt
        glob = jnp.concatenate([p1, p2], axis=1)
        he = (jnp.dot(glob, wl_ref[...], preferred_element_type=jnp.float32)
              + bl_ref[...])
        s = 0.9999950000374997
        m = jnp.maximum(
            (jnp.dot(he, w1_ref[...], preferred_element_type=jnp.float32)
             + b1_ref[...]) * s * g1_ref[...] + be1_ref[...], 0.0)
        m = jnp.maximum(
            (jnp.dot(m, w2_ref[...], preferred_element_type=jnp.float32)
             + b2_ref[...]) * s * g2_ref[...] + be2_ref[...], 0.0)
        out_ref[...] = jnp.dot(m, wo_ref[...], preferred_element_type=jnp.float32)


def _full(shape):
    return pl.BlockSpec(shape, lambda i: tuple(0 for _ in shape))


_prep = pl.pallas_call(
    _prep_body,
    grid=(NRB,),
    in_specs=[
        pl.BlockSpec((RB, D), lambda i: (i, 0)),
        _full((D, D)),
        _full((1, D)),
        _full((D, D)),
    ],
    out_specs=pl.BlockSpec((RB, D), lambda i: (i, 0)),
    out_shape=jax.ShapeDtypeStruct((N_NODES, D), jnp.float32),
)

_combine = pl.pallas_call(
    _combine_body,
    grid=(NRB,),
    in_specs=[
        pl.BlockSpec((NC, RB, D), lambda i: (0, i, 0)),
        pl.BlockSpec((NC, RB, CW), lambda i: (0, i, 0)),
    ],
    out_specs=pl.BlockSpec((RB, D), lambda i: (i, 0)),
    out_shape=jax.ShapeDtypeStruct((N_NODES, D), jnp.float32),
)

_post1 = pl.pallas_call(
    _post1_body,
    grid=(NRB,),
    in_specs=[
        pl.BlockSpec((NC, RB, D), lambda i: (0, i, 0)),
        pl.BlockSpec((NC, RB, CW), lambda i: (0, i, 0)),
        _full((1, D)),
        _full((D, D)),
        pl.BlockSpec((1, 1, RB), lambda i: (i, 0, 0)),
    ],
    out_specs=(
        pl.BlockSpec((RB, D), lambda i: (i, 0)),
        pl.BlockSpec((G, D), lambda i: (0, 0)),
        pl.BlockSpec((G, 1), lambda i: (0, 0)),
    ),
    out_shape=(
        jax.ShapeDtypeStruct((N_NODES, D), jnp.float32),
        jax.ShapeDtypeStruct((G, D), jnp.float32),
        jax.ShapeDtypeStruct((G, 1), jnp.float32),
    ),
)

_post2 = pl.pallas_call(
    _post2_body,
    grid=(NRB,),
    in_specs=[
        pl.BlockSpec((NC, RB, D), lambda i: (0, i, 0)),
        pl.BlockSpec((NC, RB, CW), lambda i: (0, i, 0)),
        _full((1, D)),
        pl.BlockSpec((1, 1, RB), lambda i: (i, 0, 0)),
        _full((G, D)),
        _full((G, 1)),
        _full((256, D)),
        _full((1, D)),
        _full((D, 64)),
        _full((1, 64)),
        _full((1, 64)),
        _full((1, 64)),
        _full((64, 32)),
        _full((1, 32)),
        _full((1, 32)),
        _full((1, 32)),
        _full((32, 4)),
    ],
    out_specs=(
        pl.BlockSpec((G, D), lambda i: (0, 0)),
        pl.BlockSpec((G, 4), lambda i: (0, 0)),
    ),
    out_shape=(
        jax.ShapeDtypeStruct((G, D), jnp.float32),
        jax.ShapeDtypeStruct((G, 4), jnp.float32),
    ),
)


def kernel(x, edge_index, batch, W_first, b_first, Wc1, bc1, Wc2, bc2, W_lin,
           b_lin, Wm1, bm1, g1, be1, Wm2, bm2, g2, be2, Wout):
    f32 = jnp.float32
    node_idx = edge_index[0]
    hedge_idx = edge_index[1]

    npad = NNZ_P - NNZ
    pad_g = (jnp.arange(npad, dtype=jnp.int32) * 41) % N_NODES
    pad_s = N_HEDGES + jnp.arange(npad, dtype=jnp.int32) % (NP - N_HEDGES)
    gidx1 = jnp.concatenate([node_idx, pad_g]).reshape(NW, NBLK, K)
    sidx1 = jnp.concatenate([hedge_idx, pad_s]).reshape(NW, NBLK, K)
    gidx2 = jnp.concatenate([hedge_idx, pad_g]).reshape(NW, NBLK, K)
    sidx2 = jnp.concatenate([node_idx, pad_s]).reshape(NW, NBLK, K)
    idx1 = jnp.stack([gidx1, sidx1], axis=2)
    idx2 = jnp.stack([gidx2, sidx2], axis=2)

    zeros = jnp.zeros((RPT, D), f32)
    zeros_c = jnp.zeros((RPT, CW), f32)
    ones = jnp.ones((K, CW), f32)
    batch_r = batch.reshape(NRB, 1, RB)

    sc_c = _make_sc_pass(True)
    sc_n = _make_sc_pass(False)
    y1 = _prep(x, W_first, b_first.reshape(1, D), Wc1)
    pm1, cb = sc_c(y1, idx1, zeros, zeros_c, ones)
    m1 = _combine(pm1, cb)
    po1, cd = sc_c(m1, idx2, zeros, zeros_c, ones)
    y2, pool1, cntg = _post1(po1, cd, bc1.reshape(1, D), Wc2, batch_r)
    pm2 = sc_n(y2, idx1, zeros, zeros_c, ones)
    m2 = _combine(pm2, cb)
    po2 = sc_n(m2, idx2, zeros, zeros_c, ones)
    _pool2, out = _post2(po2, cd, bc2.reshape(1, D), batch_r, pool1, cntg,
                         W_lin, b_lin.reshape(1, D), Wm1, bm1.reshape(1, 64),
                         g1.reshape(1, 64), be1.reshape(1, 64), Wm2,
                         bm2.reshape(1, 32), g2.reshape(1, 32),
                         be2.reshape(1, 32), Wout)
    return out
